# baseline (speedup 1.0000x reference)
import functools

import jax
import jax.numpy as jnp
from jax.experimental import pallas as pl
from jax.experimental.pallas import tpu as pltpu

N, D = 4096, 128
BM = 512
NT = N // BM


def _fused_kernel(eps_ref, adj_ref, xb_ref, g_ref, wx_ref, wa_ref, bmat_ref,
                  out_ref, ax_ref):
    i = pl.program_id(0)
    f32 = jnp.float32

    @pl.when(i > 0)
    def _experts():
        j = i - 1
        ax = ax_ref[...]
        xt = xb_ref[pl.ds(j * BM, BM), :]
        gv = g_ref[...]
        p = jnp.dot(xt, wx_ref[...], preferred_element_type=f32)
        q = jnp.dot(ax.astype(jnp.bfloat16), wa_ref[...],
                    preferred_element_type=f32)
        out = (gv[:, 0:1] * p[:, 0:D]
               + gv[:, 1:2] * (p[:, D:2 * D] + q[:, 0:D])
               + gv[:, 2:3] * ((1.0 + eps_ref[0]) * p[:, 2 * D:3 * D]
                               + q[:, D:2 * D])
               + gv[:, 3:4] * q[:, 2 * D:3 * D]
               + jnp.dot(gv, bmat_ref[...], preferred_element_type=f32))
        out_ref[...] = out

    @pl.when(i < NT)
    def _propagate():
        ax_ref[...] = jnp.dot(adj_ref[...].astype(jnp.bfloat16), xb_ref[...],
                              preferred_element_type=f32)


@functools.partial(jax.jit, static_argnames=("interpret",))
def _run(x, adj, g, eps_gin, W_tag0, W_tag1, W_gin, W_gcn, bmat,
         interpret=False):
    eps = jnp.asarray(eps_gin, jnp.float32).reshape((1,))
    bf16 = jnp.bfloat16
    xb = x.astype(bf16)
    wx = jnp.concatenate([W_tag0, W_tag1[:D, :], W_gin], axis=1).astype(bf16)
    wa = jnp.concatenate([W_tag1[D:, :], W_gin, W_gcn], axis=1).astype(bf16)
    full = lambda shape: pl.BlockSpec(shape, lambda i: (0, 0))
    prev = lambda i: (jnp.maximum(i - 1, 0), 0)
    return pl.pallas_call(
        _fused_kernel,
        grid=(NT + 1,),
        in_specs=[
            pl.BlockSpec(memory_space=pltpu.SMEM),
            pl.BlockSpec((BM, N), lambda i: (jnp.minimum(i, NT - 1), 0)),
            full((N, D)),
            pl.BlockSpec((BM, 4), prev),
            full((D, 3 * D)),
            full((D, 3 * D)),
            full((4, D)),
        ],
        out_specs=pl.BlockSpec((BM, D), prev),
        out_shape=jax.ShapeDtypeStruct((N, D), jnp.float32),
        scratch_shapes=[pltpu.VMEM((BM, D), jnp.float32)],
        interpret=interpret,
    )(eps, adj, xb, g, wx, wa, bmat)


def kernel(x, adj, g, dropout, W_tag0, b_tag0, W_tag1, b_tag1, W_gin, b_gin,
           eps_gin, W_gcn, b_gcn):
    bmat = jnp.stack([b_tag0, b_tag1, b_gin, b_gcn], axis=0)
    return _run(x, adj, g, eps_gin, W_tag0, W_tag1, W_gin, W_gcn, bmat)


# speedup vs baseline: 1.1242x; 1.1242x over previous
import functools

import jax
import jax.numpy as jnp
from jax.experimental import pallas as pl
from jax.experimental.pallas import tpu as pltpu

N, D = 4096, 128
BM = 512
NT = N // BM
_DN = (((1,), (0,)), ((), ()))


def _fused_kernel(eps_ref, adj_ref, x_ref, g_ref, w0_ref, w1x_ref, w1a_ref,
                  wgin_ref, wgcn_ref, bmat_ref, out_ref, ax_ref):
    i = pl.program_id(0)
    f32 = jnp.float32
    bf16 = jnp.bfloat16

    @pl.when(i > 0)
    def _experts():
        j = i - 1
        ax = ax_ref[...]
        xt = x_ref[pl.ds(j * BM, BM), :].astype(bf16)
        gv = g_ref[...]
        axb = ax.astype(bf16)
        ub = ((1.0 + eps_ref[0]) * xt.astype(f32) + ax).astype(bf16)
        dot = lambda a, b: jnp.dot(a, b.astype(bf16), preferred_element_type=f32)
        out = (gv[:, 0:1] * dot(xt, w0_ref[...])
               + gv[:, 1:2] * (dot(xt, w1x_ref[...]) + dot(axb, w1a_ref[...]))
               + gv[:, 2:3] * dot(ub, wgin_ref[...])
               + gv[:, 3:4] * dot(axb, wgcn_ref[...])
               + jnp.dot(gv, bmat_ref[...], preferred_element_type=f32))
        out_ref[...] = out

    @pl.when(i < NT)
    def _propagate():
        xb = x_ref[...].astype(bf16)
        ax_ref[...] = jax.lax.dot_general(adj_ref[...], xb, _DN,
                                          preferred_element_type=f32)


@functools.partial(jax.jit, static_argnames=("interpret",))
def _run(x, adj, g, eps_gin, W_tag0, W_tag1, W_gin, W_gcn, bmat,
         interpret=False):
    eps = jnp.asarray(eps_gin, jnp.float32).reshape((1,))
    W1x = W_tag1[:D, :]
    W1a = W_tag1[D:, :]
    full = lambda shape: pl.BlockSpec(shape, lambda i: (0, 0))
    prev = lambda i: (jnp.maximum(i - 1, 0), 0)
    return pl.pallas_call(
        _fused_kernel,
        grid=(NT + 1,),
        in_specs=[
            pl.BlockSpec(memory_space=pltpu.SMEM),
            pl.BlockSpec((BM, N), lambda i: (jnp.minimum(i, NT - 1), 0)),
            full((N, D)),
            pl.BlockSpec((BM, 4), prev),
            full((D, D)), full((D, D)), full((D, D)),
            full((D, D)), full((D, D)),
            full((4, D)),
        ],
        out_specs=pl.BlockSpec((BM, D), prev),
        out_shape=jax.ShapeDtypeStruct((N, D), jnp.float32),
        scratch_shapes=[pltpu.VMEM((BM, D), jnp.float32)],
        interpret=interpret,
    )(eps, adj, x, g, W_tag0, W1x, W1a, W_gin, W_gcn, bmat)


def kernel(x, adj, g, dropout, W_tag0, b_tag0, W_tag1, b_tag1, W_gin, b_gin,
           eps_gin, W_gcn, b_gcn):
    bmat = jnp.stack([b_tag0, b_tag1, b_gin, b_gcn], axis=0)
    return _run(x, adj, g, eps_gin, W_tag0, W_tag1, W_gin, W_gcn, bmat)
